# baseline (speedup 1.0000x reference)
import functools

import jax
import jax.numpy as jnp
from jax import lax
from jax.experimental import pallas as pl
from jax.experimental.pallas import tpu as pltpu
from jax.experimental.pallas import tpu_sc as plsc

NC = 2
NS = 16
NW = NC * NS
C = 128
SLOPE = 0.01
EPS = 1e-5
NG = 64
BN = 2000


def _leaky(h):
    return jnp.where(h >= 0, h, SLOPE * h)


def _mesh():
    return plsc.VectorSubcoreMesh(core_axis_name="c", subcore_axis_name="s")


_SC_PARAMS = dict(
    compiler_params=pltpu.CompilerParams(
        needs_layout_passes=False, use_tc_tiling_on_sc=False),
)


@functools.lru_cache(maxsize=None)
def _deg_kernel(n, ep):
    epw = ep // NW

    @functools.partial(
        pl.kernel,
        out_type=jax.ShapeDtypeStruct((NW, n), jnp.float32),
        mesh=_mesh(),
        scratch_types=[
            pltpu.VMEM((epw,), jnp.int32),
            pltpu.VMEM((epw,), jnp.float32),
            pltpu.VMEM((n,), jnp.float32),
        ],
        **_SC_PARAMS,
    )
    def deg(dst_hbm, ew_hbm, out_hbm, dst_v, ew_v, acc_v):
        cid = lax.axis_index("c")
        sid = lax.axis_index("s")
        wid = cid * NS + sid
        z = jnp.zeros((16,), jnp.float32)

        def zbody(i, _):
            acc_v[pl.ds(i * 16, 16)] = z
            return 0

        lax.fori_loop(0, n // 16, zbody, 0, unroll=8)
        pltpu.sync_copy(dst_hbm.at[pl.ds(wid * epw, epw)], dst_v)
        pltpu.sync_copy(ew_hbm.at[pl.ds(wid * epw, epw)], ew_v)

        def body(i, _):
            idx = dst_v[pl.ds(i * 16, 16)]
            w = ew_v[pl.ds(i * 16, 16)]
            plsc.addupdate_scatter(acc_v, [idx], w)
            return 0

        lax.fori_loop(0, epw // 16, body, 0, unroll=4)
        pltpu.sync_copy(acc_v, out_hbm.at[wid])

    return deg


@functools.lru_cache(maxsize=None)
def _agg_kernel(n, ep, h, parts):
    npass = parts // 2
    nchunk = ep // (NS * C)
    assert nchunk % 4 == 0 and nchunk >= 8
    ngroups = nchunk // 4
    ept = ep // NS
    nzr = n // NS
    unroll = 4 if h >= 64 else 8

    @functools.partial(
        pl.kernel,
        out_type=jax.ShapeDtypeStruct((parts * n, h), jnp.float32),
        mesh=_mesh(),
        scratch_types=[
            pltpu.VMEM((nchunk, C), jnp.int32),
            pltpu.VMEM((nchunk, C), jnp.int32),
            pltpu.VMEM((ept,), jnp.float32),
            pltpu.VMEM((C, h), jnp.float32),
            pltpu.VMEM((C, h), jnp.float32),
            pltpu.VMEM((C, h), jnp.float32),
            pltpu.VMEM((C, h), jnp.float32),
            pltpu.VMEM_SHARED((n, h), jnp.float32),
            pltpu.SemaphoreType.DMA,
            pltpu.SemaphoreType.DMA,
            pltpu.SemaphoreType.DMA,
            pltpu.SemaphoreType.DMA,
            pltpu.SemaphoreType.DMA,
            pltpu.SemaphoreType.DMA,
            pltpu.SemaphoreType.DMA,
            pltpu.SemaphoreType.DMA,
        ],
        **_SC_PARAMS,
    )
    def agg(y_hbm, src_hbm, dst_hbm, ew_hbm, zeros_hbm, out_hbm,
            src_v, dst_v, ew_v, r0, r1, r2, r3, acc_sh,
            g0, g1, g2, g3, s0, s1, s2, s3):
        rows = [r0, r1, r2, r3]
        gsem = [g0, g1, g2, g3]
        ssem = [s0, s1, s2, s3]
        cid = lax.axis_index("c")
        sid = lax.axis_index("s")
        pltpu.sync_copy(src_hbm.at[pl.ds(sid * nchunk, nchunk)], src_v)
        pltpu.sync_copy(dst_hbm.at[pl.ds(sid * nchunk, nchunk)], dst_v)
        pltpu.sync_copy(ew_hbm.at[pl.ds(sid * ept, ept)], ew_v)

        def shift_src(delta):
            dvec = jnp.full((16,), delta, jnp.int32)

            def shift(r, _):
                for j in range(C // 16):
                    sl = pl.ds(j * 16, 16)
                    src_v[r, sl] = src_v[r, sl] + dvec
                return 0

            lax.fori_loop(0, nchunk, shift, 0, unroll=2)

        def gather_start(c, b):
            pltpu.make_async_copy(y_hbm.at[src_v.at[c]], rows[b],
                                  gsem[b]).start()

        def gather_wait(c, b):
            pltpu.make_async_copy(y_hbm.at[src_v.at[c]], rows[b],
                                  gsem[b]).wait()

        def scatter_start(c, b):
            pltpu.make_async_copy(rows[b], acc_sh.at[dst_v.at[c]],
                                  ssem[b]).start(add=True)

        def scatter_wait(c, b):
            pltpu.make_async_copy(rows[b], acc_sh.at[dst_v.at[c]],
                                  ssem[b]).wait()

        def multiply(c, b):
            rbuf = rows[b]

            def mbody(e, _):
                w = plsc.load_gather(
                    ew_v, [jnp.full((16,), c * C + e, jnp.int32)])
                for j in range(h // 16):
                    sl = pl.ds(j * 16, 16)
                    rbuf[e, sl] = rbuf[e, sl] * w
                return 0

            lax.fori_loop(0, C, mbody, 0, unroll=unroll)

        def position(c, k, pre_c, drain):
            pb = (k + 2) % 4
            if pre_c is not None:
                if drain:
                    scatter_wait(pre_c - 4, pb)
                gather_start(pre_c, pb)
            gather_wait(c, k)
            multiply(c, k)
            scatter_start(c, k)

        for q in range(npass):
            shift_src(cid * npass * n if q == 0 else n)
            pltpu.sync_copy(zeros_hbm.at[pl.ds(sid * nzr, nzr)],
                            acc_sh.at[pl.ds(sid * nzr, nzr)])
            plsc.subcore_barrier()

            gather_start(0, 0)
            gather_start(1, 1)
            position(0, 0, 2, False)
            position(1, 1, 3, False)
            position(2, 2, 4, True)
            position(3, 3, 5, True)

            def gbody(g, _):
                c0 = g * 4
                for k in range(4):
                    position(c0 + k, k, c0 + k + 2, True)
                return 0

            lax.fori_loop(1, ngroups - 1, gbody, 0)
            c0 = (ngroups - 1) * 4
            position(c0 + 0, 0, c0 + 2, True)
            position(c0 + 1, 1, c0 + 3, True)
            position(c0 + 2, 2, None, False)
            position(c0 + 3, 3, None, False)
            for k in range(4):
                scatter_wait(c0 + k, k)
            plsc.subcore_barrier()
            tq = cid * npass + q
            pltpu.sync_copy(acc_sh.at[pl.ds(sid * nzr, nzr)],
                            out_hbm.at[pl.ds(tq * n + sid * nzr, nzr)])

    return agg


@functools.lru_cache(maxsize=None)
def _y1_kernel(n, fin, fo, po, nwp):
    nb = n // BN
    hn = fo // po

    def body(x_ref, w_ref, dn_ref, y_ref):
        dinv = lax.rsqrt(jnp.sum(dn_ref[...], axis=1, keepdims=True) + 1.0)
        y = jnp.dot(x_ref[...], w_ref[...],
                    preferred_element_type=jnp.float32) * dinv
        for t in range(po):
            y_ref[t] = y[:, t * hn:(t + 1) * hn]

    return pl.pallas_call(
        body,
        grid=(nb,),
        in_specs=[
            pl.BlockSpec((BN, fin), lambda i: (i, 0)),
            pl.BlockSpec((fin, fo), lambda i: (0, 0)),
            pl.BlockSpec((BN, nwp), lambda i: (i, 0)),
        ],
        out_specs=pl.BlockSpec((po, BN, hn), lambda i: (0, i, 0)),
        out_shape=jax.ShapeDtypeStruct((po, n, hn), jnp.float32),
    )


def _part_specs(nb, pi, hp):
    def mk(t):
        return pl.BlockSpec((BN, hp), lambda i, t=t: (i + t * nb, 0))

    return [mk(t) for t in range(pi)]


@functools.lru_cache(maxsize=None)
def _ymid_kernel(n, fp, fn, pi, po, nwp):
    nb = n // BN
    hp = fp // pi
    hn = fn // po

    def body(*refs):
        a = refs[:pi]
        y = refs[pi:2 * pi]
        dn, w = refs[2 * pi], refs[2 * pi + 1]
        gc = refs[2 * pi + 2:3 * pi + 2]
        bc = refs[3 * pi + 2:4 * pi + 2]
        yo = refs[4 * pi + 2]
        dinv = lax.rsqrt(jnp.sum(dn[...], axis=1, keepdims=True) + 1.0)
        wv = w[...]
        acc = None
        for t in range(pi):
            ht = _leaky(gc[t][...] * (dinv * (a[t][...] + y[t][...]))
                        + bc[t][...])
            d = jnp.dot(ht, wv[t * hp:(t + 1) * hp],
                        preferred_element_type=jnp.float32)
            acc = d if acc is None else acc + d
        yn = acc * dinv
        for t in range(po):
            yo[t] = yn[:, t * hn:(t + 1) * hn]

    return pl.pallas_call(
        body,
        grid=(nb,),
        in_specs=(
            _part_specs(nb, pi, hp) * 2
            + [
                pl.BlockSpec((BN, nwp), lambda i: (i, 0)),
                pl.BlockSpec((fp, fn), lambda i: (0, 0)),
            ]
            + [pl.BlockSpec((1, hp), lambda i: (0, 0))] * (2 * pi)
        ),
        out_specs=pl.BlockSpec((po, BN, hn), lambda i: (0, i, 0)),
        out_shape=jax.ShapeDtypeStruct((po, n, hn), jnp.float32),
    )


@functools.lru_cache(maxsize=None)
def _head_kernel(n, f, pi, nwp):
    nb = n // BN
    hp = f // pi

    def body(*refs):
        a = refs[:pi]
        y = refs[pi:2 * pi]
        dn, b3 = refs[2 * pi], refs[2 * pi + 1]
        gc = refs[2 * pi + 2:3 * pi + 2]
        bc = refs[3 * pi + 2:4 * pi + 2]
        (fw1, fb1, fw2, fb2, fw3, fb3, fw4t, fb4,
         out) = refs[4 * pi + 2:4 * pi + 11]
        sums = refs[4 * pi + 11:5 * pi + 11]
        counts = refs[5 * pi + 11]
        i = pl.program_id(0)

        @pl.when(i == 0)
        def _():
            for t in range(pi):
                sums[t][...] = jnp.zeros_like(sums[t])
            counts[...] = jnp.zeros_like(counts)

        dinv = lax.rsqrt(jnp.sum(dn[...], axis=1, keepdims=True) + 1.0)
        bvec = b3[0]
        gids = lax.broadcasted_iota(jnp.int32, (NG, BN), 0)
        p = jnp.where(gids == jnp.broadcast_to(bvec, (NG, BN)), 1.0, 0.0)
        for t in range(pi):
            ht = _leaky(gc[t][...] * (dinv * (a[t][...] + y[t][...]))
                        + bc[t][...])
            sums[t][...] += jnp.dot(p, ht, preferred_element_type=jnp.float32)
        counts[...] += jnp.dot(p, jnp.ones((BN, hp), jnp.float32),
                               preferred_element_type=jnp.float32)

        @pl.when(i == nb - 1)
        def _():
            cnt = jnp.maximum(counts[...], 1.0)
            w1 = fw1[...]
            acc = None
            for t in range(pi):
                d = jnp.dot(sums[t][...] / cnt, w1[t * hp:(t + 1) * hp],
                            preferred_element_type=jnp.float32)
                acc = d if acc is None else acc + d
            a1 = _leaky(acc + fb1[...])
            a2 = _leaky(jnp.dot(a1, fw2[...],
                                preferred_element_type=jnp.float32) + fb2[...])
            a3 = _leaky(jnp.dot(a2, fw3[...],
                                preferred_element_type=jnp.float32) + fb3[...])
            r = jnp.sum(a3 * fw4t[...], axis=1, keepdims=True) + fb4[...]
            out[...] = jnp.broadcast_to(r, (NG, 128))

    return pl.pallas_call(
        body,
        grid=(nb,),
        in_specs=(
            _part_specs(nb, pi, hp) * 2
            + [
                pl.BlockSpec((BN, nwp), lambda i: (i, 0)),
                pl.BlockSpec((1, 1, BN), lambda i: (i, 0, 0)),
            ]
            + [pl.BlockSpec((1, hp), lambda i: (0, 0))] * (2 * pi)
            + [
                pl.BlockSpec((128, 64), lambda i: (0, 0)),
                pl.BlockSpec((1, 64), lambda i: (0, 0)),
                pl.BlockSpec((64, 32), lambda i: (0, 0)),
                pl.BlockSpec((1, 32), lambda i: (0, 0)),
                pl.BlockSpec((32, 16), lambda i: (0, 0)),
                pl.BlockSpec((1, 16), lambda i: (0, 0)),
                pl.BlockSpec((1, 16), lambda i: (0, 0)),
                pl.BlockSpec((1, 1), lambda i: (0, 0)),
            ]
        ),
        out_specs=pl.BlockSpec((NG, 128), lambda i: (0, 0)),
        out_shape=jax.ShapeDtypeStruct((NG, 128), jnp.float32),
        scratch_shapes=[pltpu.VMEM((NG, hp), jnp.float32)] * (pi + 1),
    )


_P1, _P2, _P3 = 2, 2, 4


def kernel(x, edge_index, edge_weight, batch, W1, b1, W2, b2, W3, b3,
           g1, be1, g2, be2, g3, be3,
           fW1, fb1, fW2, fb2, fW3, fb3, fW4, fb4):
    n, fin = x.shape
    e = edge_weight.shape[0]
    src = edge_index[0].astype(jnp.int32)
    dst = edge_index[1].astype(jnp.int32)
    ew = edge_weight.astype(jnp.float32)

    nchunk = -4 * (-e // (4 * NS * C))
    ep = NS * C * nchunk
    pad = ep - e
    if pad:
        src = jnp.concatenate([src, jnp.zeros((pad,), jnp.int32)])
        dst = jnp.concatenate([dst, jnp.zeros((pad,), jnp.int32)])
        ew = jnp.concatenate([ew, jnp.zeros((pad,), jnp.float32)])
    src2 = src.reshape(-1, C)
    dst2 = dst.reshape(-1, C)

    epd = -NW * 16 * (-e // (NW * 16))
    padd = epd - e
    dstd = jnp.concatenate([dst[:e], jnp.zeros((padd,), jnp.int32)])
    ewd = jnp.concatenate([ew[:e], jnp.zeros((padd,), jnp.float32)])
    deg_parts = _deg_kernel(n, epd)(dstd, ewd)
    dn = deg_parts.T

    s = 1.0 / jnp.sqrt(1.0 + EPS)

    def split(v, parts):
        hh = v.shape[0] // parts
        return tuple(v[t * hh:(t + 1) * hh][None, :] for t in range(parts))

    g1c = split(g1 * s, _P1)
    b1c = split(g1 * s * b1 + be1, _P1)
    g2c = split(g2 * s, _P2)
    b2c = split(g2 * s * b2 + be2, _P2)
    g3c = split(g3 * s, _P3)
    b3c = split(g3 * s * b3 + be3, _P3)

    f1, f2, f3 = W1.shape[1], W2.shape[1], W3.shape[1]
    z1 = jnp.zeros((n, f1 // _P1), jnp.float32)
    z2 = jnp.zeros((n, f2 // _P2), jnp.float32)
    z3 = jnp.zeros((n, f3 // _P3), jnp.float32)

    y1 = _y1_kernel(n, fin, f1, _P1, NW)(x, W1, dn).reshape(
        _P1 * n, f1 // _P1)
    a1 = _agg_kernel(n, ep, f1 // _P1, _P1)(y1, src2, dst2, ew, z1)
    y2 = _ymid_kernel(n, f1, f2, _P1, _P2, NW)(
        *([a1] * _P1), *([y1] * _P1), dn, W2, *g1c, *b1c
    ).reshape(_P2 * n, f2 // _P2)
    a2 = _agg_kernel(n, ep, f2 // _P2, _P2)(y2, src2, dst2, ew, z2)
    y3 = _ymid_kernel(n, f2, f3, _P2, _P3, NW)(
        *([a2] * _P2), *([y2] * _P2), dn, W3, *g2c, *b2c
    ).reshape(_P3 * n, f3 // _P3)
    a3 = _agg_kernel(n, ep, f3 // _P3, _P3)(y3, src2, dst2, ew, z3)

    nb = n // BN
    batch3 = batch.astype(jnp.int32).reshape(nb, 1, BN)
    head = _head_kernel(n, f3, _P3, NW)(
        *([a3] * _P3), *([y3] * _P3), dn, batch3, *g3c, *b3c,
        fW1, fb1[None, :], fW2, fb2[None, :], fW3, fb3[None, :],
        fW4.T, fb4[None, :])
    return head[:, 0]


# speedup vs baseline: 1.2847x; 1.2847x over previous
import functools

import jax
import jax.numpy as jnp
from jax import lax
from jax.experimental import pallas as pl
from jax.experimental.pallas import tpu as pltpu
from jax.experimental.pallas import tpu_sc as plsc

NC = 2
NS = 16
NW = NC * NS
C = 128
SLOPE = 0.01
EPS = 1e-5
NG = 64
BN = 2000


def _leaky(h):
    return jnp.where(h >= 0, h, SLOPE * h)


def _mesh():
    return plsc.VectorSubcoreMesh(core_axis_name="c", subcore_axis_name="s")


_SC_PARAMS = dict(
    compiler_params=pltpu.CompilerParams(
        needs_layout_passes=False, use_tc_tiling_on_sc=False),
)


@functools.lru_cache(maxsize=None)
def _deg_kernel(n, ep):
    epw = ep // NW

    @functools.partial(
        pl.kernel,
        out_type=jax.ShapeDtypeStruct((NW, n), jnp.float32),
        mesh=_mesh(),
        scratch_types=[
            pltpu.VMEM((epw,), jnp.int32),
            pltpu.VMEM((epw,), jnp.float32),
            pltpu.VMEM((n,), jnp.float32),
        ],
        **_SC_PARAMS,
    )
    def deg(dst_hbm, ew_hbm, out_hbm, dst_v, ew_v, acc_v):
        cid = lax.axis_index("c")
        sid = lax.axis_index("s")
        wid = cid * NS + sid
        z = jnp.zeros((16,), jnp.float32)

        def zbody(i, _):
            acc_v[pl.ds(i * 16, 16)] = z
            return 0

        lax.fori_loop(0, n // 16, zbody, 0, unroll=8)
        pltpu.sync_copy(dst_hbm.at[pl.ds(wid * epw, epw)], dst_v)
        pltpu.sync_copy(ew_hbm.at[pl.ds(wid * epw, epw)], ew_v)

        def body(i, _):
            idx = dst_v[pl.ds(i * 16, 16)]
            w = ew_v[pl.ds(i * 16, 16)]
            plsc.addupdate_scatter(acc_v, [idx], w)
            return 0

        lax.fori_loop(0, epw // 16, body, 0, unroll=4)
        pltpu.sync_copy(acc_v, out_hbm.at[wid])

    return deg


@functools.lru_cache(maxsize=None)
def _agg_kernel(n, ep, h, parts):
    npass = parts // 2
    nchunk = ep // (NS * C)
    assert nchunk % 4 == 0 and nchunk >= 8
    ngroups = nchunk // 4
    ept = ep // NS
    nzr = n // NS
    unroll = 4 if h >= 64 else 8

    @functools.partial(
        pl.kernel,
        out_type=jax.ShapeDtypeStruct((parts * n, h), jnp.float32),
        mesh=_mesh(),
        scratch_types=[
            pltpu.VMEM((nchunk, C), jnp.int32),
            pltpu.VMEM((nchunk, C), jnp.int32),
            pltpu.VMEM((ept,), jnp.float32),
            pltpu.VMEM((C, h), jnp.float32),
            pltpu.VMEM((C, h), jnp.float32),
            pltpu.VMEM((C, h), jnp.float32),
            pltpu.VMEM((C, h), jnp.float32),
            pltpu.VMEM_SHARED((n, h), jnp.float32),
            pltpu.VMEM_SHARED((n, h), jnp.float32),
            pltpu.SemaphoreType.DMA,
            pltpu.SemaphoreType.DMA,
            pltpu.SemaphoreType.DMA,
            pltpu.SemaphoreType.DMA,
            pltpu.SemaphoreType.DMA,
            pltpu.SemaphoreType.DMA,
            pltpu.SemaphoreType.DMA,
            pltpu.SemaphoreType.DMA,
        ],
        **_SC_PARAMS,
    )
    def agg(y_hbm, src_hbm, dst_hbm, ew_hbm, zeros_hbm, out_hbm,
            src_v, dst_v, ew_v, r0, r1, r2, r3, acc_sh, ystage,
            g0, g1, g2, g3, s0, s1, s2, s3):
        rows = [r0, r1, r2, r3]
        gsem = [g0, g1, g2, g3]
        ssem = [s0, s1, s2, s3]
        cid = lax.axis_index("c")
        sid = lax.axis_index("s")
        pltpu.sync_copy(src_hbm.at[pl.ds(sid * nchunk, nchunk)], src_v)
        pltpu.sync_copy(dst_hbm.at[pl.ds(sid * nchunk, nchunk)], dst_v)
        pltpu.sync_copy(ew_hbm.at[pl.ds(sid * ept, ept)], ew_v)

        def gather_start(c, b):
            pltpu.make_async_copy(ystage.at[src_v.at[c]], rows[b],
                                  gsem[b]).start()

        def gather_wait(c, b):
            pltpu.make_async_copy(ystage.at[src_v.at[c]], rows[b],
                                  gsem[b]).wait()

        def scatter_start(c, b):
            pltpu.make_async_copy(rows[b], acc_sh.at[dst_v.at[c]],
                                  ssem[b]).start(add=True)

        def scatter_wait(c, b):
            pltpu.make_async_copy(rows[b], acc_sh.at[dst_v.at[c]],
                                  ssem[b]).wait()

        def multiply(c, b):
            rbuf = rows[b]

            def mbody(e, _):
                w = plsc.load_gather(
                    ew_v, [jnp.full((16,), c * C + e, jnp.int32)])
                for j in range(h // 16):
                    sl = pl.ds(j * 16, 16)
                    rbuf[e, sl] = rbuf[e, sl] * w
                return 0

            lax.fori_loop(0, C, mbody, 0, unroll=unroll)

        def position(c, k, pre_c, drain):
            pb = (k + 2) % 4
            if pre_c is not None:
                if drain:
                    scatter_wait(pre_c - 4, pb)
                gather_start(pre_c, pb)
            gather_wait(c, k)
            multiply(c, k)
            scatter_start(c, k)

        for q in range(npass):
            tq = cid * npass + q
            pltpu.sync_copy(zeros_hbm.at[pl.ds(sid * nzr, nzr)],
                            acc_sh.at[pl.ds(sid * nzr, nzr)])
            pltpu.sync_copy(y_hbm.at[pl.ds(tq * n + sid * nzr, nzr)],
                            ystage.at[pl.ds(sid * nzr, nzr)])
            plsc.subcore_barrier()

            gather_start(0, 0)
            gather_start(1, 1)
            position(0, 0, 2, False)
            position(1, 1, 3, False)
            position(2, 2, 4, True)
            position(3, 3, 5, True)

            def gbody(g, _):
                c0 = g * 4
                for k in range(4):
                    position(c0 + k, k, c0 + k + 2, True)
                return 0

            lax.fori_loop(1, ngroups - 1, gbody, 0)
            c0 = (ngroups - 1) * 4
            position(c0 + 0, 0, c0 + 2, True)
            position(c0 + 1, 1, c0 + 3, True)
            position(c0 + 2, 2, None, False)
            position(c0 + 3, 3, None, False)
            for k in range(4):
                scatter_wait(c0 + k, k)
            plsc.subcore_barrier()
            pltpu.sync_copy(acc_sh.at[pl.ds(sid * nzr, nzr)],
                            out_hbm.at[pl.ds(tq * n + sid * nzr, nzr)])

    return agg


@functools.lru_cache(maxsize=None)
def _y1_kernel(n, fin, fo, po, nwp):
    nb = n // BN
    hn = fo // po

    def body(x_ref, w_ref, dn_ref, y_ref):
        dinv = 1.0 / jnp.sqrt(jnp.sum(dn_ref[...], axis=1, keepdims=True) + 1.0)
        y = jnp.dot(x_ref[...], w_ref[...],
                    preferred_element_type=jnp.float32) * dinv
        for t in range(po):
            y_ref[t] = y[:, t * hn:(t + 1) * hn]

    return pl.pallas_call(
        body,
        grid=(nb,),
        in_specs=[
            pl.BlockSpec((BN, fin), lambda i: (i, 0)),
            pl.BlockSpec((fin, fo), lambda i: (0, 0)),
            pl.BlockSpec((BN, nwp), lambda i: (i, 0)),
        ],
        out_specs=pl.BlockSpec((po, BN, hn), lambda i: (0, i, 0)),
        out_shape=jax.ShapeDtypeStruct((po, n, hn), jnp.float32),
    )


def _part_specs(nb, pi, hp):
    def mk(t):
        return pl.BlockSpec((BN, hp), lambda i, t=t: (i + t * nb, 0))

    return [mk(t) for t in range(pi)]


@functools.lru_cache(maxsize=None)
def _ymid_kernel(n, fp, fn, pi, po, nwp):
    nb = n // BN
    hp = fp // pi
    hn = fn // po

    def body(*refs):
        a = refs[:pi]
        y = refs[pi:2 * pi]
        dn, w = refs[2 * pi], refs[2 * pi + 1]
        gc = refs[2 * pi + 2:3 * pi + 2]
        bc = refs[3 * pi + 2:4 * pi + 2]
        yo = refs[4 * pi + 2]
        dinv = 1.0 / jnp.sqrt(jnp.sum(dn[...], axis=1, keepdims=True) + 1.0)
        wv = w[...]
        acc = None
        for t in range(pi):
            ht = _leaky(gc[t][...] * (dinv * (a[t][...] + y[t][...]))
                        + bc[t][...])
            d = jnp.dot(ht, wv[t * hp:(t + 1) * hp],
                        preferred_element_type=jnp.float32)
            acc = d if acc is None else acc + d
        yn = acc * dinv
        for t in range(po):
            yo[t] = yn[:, t * hn:(t + 1) * hn]

    return pl.pallas_call(
        body,
        grid=(nb,),
        in_specs=(
            _part_specs(nb, pi, hp) * 2
            + [
                pl.BlockSpec((BN, nwp), lambda i: (i, 0)),
                pl.BlockSpec((fp, fn), lambda i: (0, 0)),
            ]
            + [pl.BlockSpec((1, hp), lambda i: (0, 0))] * (2 * pi)
        ),
        out_specs=pl.BlockSpec((po, BN, hn), lambda i: (0, i, 0)),
        out_shape=jax.ShapeDtypeStruct((po, n, hn), jnp.float32),
    )


@functools.lru_cache(maxsize=None)
def _head_kernel(n, f, pi, nwp):
    nb = n // BN
    hp = f // pi

    def body(*refs):
        a = refs[:pi]
        y = refs[pi:2 * pi]
        dn, b3 = refs[2 * pi], refs[2 * pi + 1]
        gc = refs[2 * pi + 2:3 * pi + 2]
        bc = refs[3 * pi + 2:4 * pi + 2]
        (fw1, fb1, fw2, fb2, fw3, fb3, fw4t, fb4,
         out) = refs[4 * pi + 2:4 * pi + 11]
        sums = refs[4 * pi + 11:5 * pi + 11]
        counts = refs[5 * pi + 11]
        i = pl.program_id(0)

        @pl.when(i == 0)
        def _():
            for t in range(pi):
                sums[t][...] = jnp.zeros_like(sums[t])
            counts[...] = jnp.zeros_like(counts)

        dinv = 1.0 / jnp.sqrt(jnp.sum(dn[...], axis=1, keepdims=True) + 1.0)
        bvec = b3[0]
        gids = lax.broadcasted_iota(jnp.int32, (NG, BN), 0)
        p = jnp.where(gids == jnp.broadcast_to(bvec, (NG, BN)), 1.0, 0.0)
        for t in range(pi):
            ht = _leaky(gc[t][...] * (dinv * (a[t][...] + y[t][...]))
                        + bc[t][...])
            sums[t][...] += jnp.dot(p, ht,
                                    preferred_element_type=jnp.float32,
                                    precision=lax.Precision.HIGHEST)
        counts[...] += jnp.dot(p, jnp.ones((BN, hp), jnp.float32),
                               preferred_element_type=jnp.float32,
                               precision=lax.Precision.HIGHEST)

        @pl.when(i == nb - 1)
        def _():
            cnt = jnp.maximum(counts[...], 1.0)
            w1 = fw1[...]
            acc = None
            for t in range(pi):
                d = jnp.dot(sums[t][...] / cnt, w1[t * hp:(t + 1) * hp],
                            preferred_element_type=jnp.float32)
                acc = d if acc is None else acc + d
            a1 = _leaky(acc + fb1[...])
            a2 = _leaky(jnp.dot(a1, fw2[...],
                                preferred_element_type=jnp.float32) + fb2[...])
            a3 = _leaky(jnp.dot(a2, fw3[...],
                                preferred_element_type=jnp.float32) + fb3[...])
            r = jnp.dot(a3, fw4t[...],
                        preferred_element_type=jnp.float32) + fb4[...]
            out[...] = jnp.broadcast_to(r, (NG, 128))

    return pl.pallas_call(
        body,
        grid=(nb,),
        in_specs=(
            _part_specs(nb, pi, hp) * 2
            + [
                pl.BlockSpec((BN, nwp), lambda i: (i, 0)),
                pl.BlockSpec((1, 1, BN), lambda i: (i, 0, 0)),
            ]
            + [pl.BlockSpec((1, hp), lambda i: (0, 0))] * (2 * pi)
            + [
                pl.BlockSpec((128, 64), lambda i: (0, 0)),
                pl.BlockSpec((1, 64), lambda i: (0, 0)),
                pl.BlockSpec((64, 32), lambda i: (0, 0)),
                pl.BlockSpec((1, 32), lambda i: (0, 0)),
                pl.BlockSpec((32, 16), lambda i: (0, 0)),
                pl.BlockSpec((1, 16), lambda i: (0, 0)),
                pl.BlockSpec((16, 1), lambda i: (0, 0)),
                pl.BlockSpec((1, 1), lambda i: (0, 0)),
            ]
        ),
        out_specs=pl.BlockSpec((NG, 128), lambda i: (0, 0)),
        out_shape=jax.ShapeDtypeStruct((NG, 128), jnp.float32),
        scratch_shapes=[pltpu.VMEM((NG, hp), jnp.float32)] * (pi + 1),
    )


_P1, _P2, _P3 = 2, 2, 4


def kernel(x, edge_index, edge_weight, batch, W1, b1, W2, b2, W3, b3,
           g1, be1, g2, be2, g3, be3,
           fW1, fb1, fW2, fb2, fW3, fb3, fW4, fb4):
    n, fin = x.shape
    e = edge_weight.shape[0]
    src = edge_index[0].astype(jnp.int32)
    dst = edge_index[1].astype(jnp.int32)
    ew = edge_weight.astype(jnp.float32)

    nchunk = -4 * (-e // (4 * NS * C))
    ep = NS * C * nchunk
    pad = ep - e
    if pad:
        src = jnp.concatenate([src, jnp.zeros((pad,), jnp.int32)])
        dst = jnp.concatenate([dst, jnp.zeros((pad,), jnp.int32)])
        ew = jnp.concatenate([ew, jnp.zeros((pad,), jnp.float32)])
    src2 = src.reshape(-1, C)
    dst2 = dst.reshape(-1, C)

    epd = -NW * 16 * (-e // (NW * 16))
    padd = epd - e
    dstd = jnp.concatenate([dst[:e], jnp.zeros((padd,), jnp.int32)])
    ewd = jnp.concatenate([ew[:e], jnp.zeros((padd,), jnp.float32)])
    deg_parts = _deg_kernel(n, epd)(dstd, ewd)
    dn = deg_parts.T

    s = 1.0 / jnp.sqrt(1.0 + EPS)

    def split(v, parts):
        hh = v.shape[0] // parts
        return tuple(v[t * hh:(t + 1) * hh][None, :] for t in range(parts))

    g1c = split(g1 * s, _P1)
    b1c = split(g1 * s * b1 + be1, _P1)
    g2c = split(g2 * s, _P2)
    b2c = split(g2 * s * b2 + be2, _P2)
    g3c = split(g3 * s, _P3)
    b3c = split(g3 * s * b3 + be3, _P3)

    f1, f2, f3 = W1.shape[1], W2.shape[1], W3.shape[1]
    z1 = jnp.zeros((n, f1 // _P1), jnp.float32)
    z2 = jnp.zeros((n, f2 // _P2), jnp.float32)
    z3 = jnp.zeros((n, f3 // _P3), jnp.float32)

    y1 = _y1_kernel(n, fin, f1, _P1, NW)(x, W1, dn).reshape(
        _P1 * n, f1 // _P1)
    a1 = _agg_kernel(n, ep, f1 // _P1, _P1)(y1, src2, dst2, ew, z1)
    y2 = _ymid_kernel(n, f1, f2, _P1, _P2, NW)(
        *([a1] * _P1), *([y1] * _P1), dn, W2, *g1c, *b1c
    ).reshape(_P2 * n, f2 // _P2)
    a2 = _agg_kernel(n, ep, f2 // _P2, _P2)(y2, src2, dst2, ew, z2)
    y3 = _ymid_kernel(n, f2, f3, _P2, _P3, NW)(
        *([a2] * _P2), *([y2] * _P2), dn, W3, *g2c, *b2c
    ).reshape(_P3 * n, f3 // _P3)
    a3 = _agg_kernel(n, ep, f3 // _P3, _P3)(y3, src2, dst2, ew, z3)

    nb = n // BN
    batch3 = batch.astype(jnp.int32).reshape(nb, 1, BN)
    head = _head_kernel(n, f3, _P3, NW)(
        *([a3] * _P3), *([y3] * _P3), dn, batch3, *g3c, *b3c,
        fW1, fb1[None, :], fW2, fb2[None, :], fW3, fb3[None, :],
        fW4, fb4[None, :])
    return head[:, 0]


# speedup vs baseline: 1.3872x; 1.0798x over previous
import functools

import jax
import jax.numpy as jnp
from jax import lax
from jax.experimental import pallas as pl
from jax.experimental.pallas import tpu as pltpu
from jax.experimental.pallas import tpu_sc as plsc

NC = 2
NS = 16
NW = NC * NS
C = 128
SLOPE = 0.01
EPS = 1e-5
NG = 64
BN = 2000


def _leaky(h):
    return jnp.where(h >= 0, h, SLOPE * h)


def _mesh():
    return plsc.VectorSubcoreMesh(core_axis_name="c", subcore_axis_name="s")


_SC_PARAMS = dict(
    compiler_params=pltpu.CompilerParams(
        needs_layout_passes=False, use_tc_tiling_on_sc=False),
)


@functools.lru_cache(maxsize=None)
def _deg_kernel(n, ep):
    epw = ep // NW

    @functools.partial(
        pl.kernel,
        out_type=jax.ShapeDtypeStruct((NW, n), jnp.float32),
        mesh=_mesh(),
        scratch_types=[
            pltpu.VMEM((epw,), jnp.int32),
            pltpu.VMEM((epw,), jnp.float32),
            pltpu.VMEM((n,), jnp.float32),
        ],
        **_SC_PARAMS,
    )
    def deg(dst_hbm, ew_hbm, out_hbm, dst_v, ew_v, acc_v):
        cid = lax.axis_index("c")
        sid = lax.axis_index("s")
        wid = cid * NS + sid
        z = jnp.zeros((16,), jnp.float32)

        def zbody(i, _):
            acc_v[pl.ds(i * 16, 16)] = z
            return 0

        lax.fori_loop(0, n // 16, zbody, 0, unroll=8)
        pltpu.sync_copy(dst_hbm.at[pl.ds(wid * epw, epw)], dst_v)
        pltpu.sync_copy(ew_hbm.at[pl.ds(wid * epw, epw)], ew_v)

        def body(i, _):
            idx = dst_v[pl.ds(i * 16, 16)]
            w = ew_v[pl.ds(i * 16, 16)]
            plsc.addupdate_scatter(acc_v, [idx], w)
            return 0

        lax.fori_loop(0, epw // 16, body, 0, unroll=4)
        pltpu.sync_copy(acc_v, out_hbm.at[wid])

    return deg


@functools.lru_cache(maxsize=None)
def _agg_kernel(n, ep, h, parts):
    edge_split = parts == 1
    npass = max(parts // 2, 1)
    ntile = NW if edge_split else NS
    nchunk = ep // (ntile * C)
    assert nchunk % 4 == 0 and nchunk >= 8
    ngroups = nchunk // 4
    ept = ep // ntile
    nzr = n // NS
    unroll = 4 if h >= 64 else 8
    nout = 2 if edge_split else parts

    @functools.partial(
        pl.kernel,
        out_type=jax.ShapeDtypeStruct((nout * n, h), jnp.float32),
        mesh=_mesh(),
        scratch_types=[
            pltpu.VMEM((nchunk, C), jnp.int32),
            pltpu.VMEM((nchunk, C), jnp.int32),
            pltpu.VMEM((ept,), jnp.float32),
            pltpu.VMEM((C, h), jnp.float32),
            pltpu.VMEM((C, h), jnp.float32),
            pltpu.VMEM((C, h), jnp.float32),
            pltpu.VMEM((C, h), jnp.float32),
            pltpu.VMEM_SHARED((n, h), jnp.float32),
            pltpu.VMEM_SHARED((n, h), jnp.float32),
            pltpu.SemaphoreType.DMA,
            pltpu.SemaphoreType.DMA,
            pltpu.SemaphoreType.DMA,
            pltpu.SemaphoreType.DMA,
            pltpu.SemaphoreType.DMA,
            pltpu.SemaphoreType.DMA,
            pltpu.SemaphoreType.DMA,
            pltpu.SemaphoreType.DMA,
        ],
        **_SC_PARAMS,
    )
    def agg(y_hbm, src_hbm, dst_hbm, ew_hbm, zeros_hbm, out_hbm,
            src_v, dst_v, ew_v, r0, r1, r2, r3, acc_sh, ystage,
            g0, g1, g2, g3, s0, s1, s2, s3):
        rows = [r0, r1, r2, r3]
        gsem = [g0, g1, g2, g3]
        ssem = [s0, s1, s2, s3]
        cid = lax.axis_index("c")
        sid = lax.axis_index("s")
        tix = cid * NS + sid if edge_split else sid
        pltpu.sync_copy(src_hbm.at[pl.ds(tix * nchunk, nchunk)], src_v)
        pltpu.sync_copy(dst_hbm.at[pl.ds(tix * nchunk, nchunk)], dst_v)
        pltpu.sync_copy(ew_hbm.at[pl.ds(tix * ept, ept)], ew_v)

        def gather_start(c, b):
            pltpu.make_async_copy(ystage.at[src_v.at[c]], rows[b],
                                  gsem[b]).start()

        def gather_wait(c, b):
            pltpu.make_async_copy(ystage.at[src_v.at[c]], rows[b],
                                  gsem[b]).wait()

        def scatter_start(c, b):
            pltpu.make_async_copy(rows[b], acc_sh.at[dst_v.at[c]],
                                  ssem[b]).start(add=True)

        def scatter_wait(c, b):
            pltpu.make_async_copy(rows[b], acc_sh.at[dst_v.at[c]],
                                  ssem[b]).wait()

        def multiply(c, b):
            rbuf = rows[b]

            def mbody(e, _):
                w = plsc.load_gather(
                    ew_v, [jnp.full((16,), c * C + e, jnp.int32)])
                for j in range(h // 16):
                    sl = pl.ds(j * 16, 16)
                    rbuf[e, sl] = rbuf[e, sl] * w
                return 0

            lax.fori_loop(0, C, mbody, 0, unroll=unroll)

        def position(c, k, pre_c, drain):
            pb = (k + 2) % 4
            if pre_c is not None:
                if drain:
                    scatter_wait(pre_c - 4, pb)
                gather_start(pre_c, pb)
            gather_wait(c, k)
            multiply(c, k)
            scatter_start(c, k)

        for q in range(npass):
            tq = cid * npass + q
            ty = 0 if edge_split else tq
            pltpu.sync_copy(zeros_hbm.at[pl.ds(sid * nzr, nzr)],
                            acc_sh.at[pl.ds(sid * nzr, nzr)])
            pltpu.sync_copy(y_hbm.at[pl.ds(ty * n + sid * nzr, nzr)],
                            ystage.at[pl.ds(sid * nzr, nzr)])
            plsc.subcore_barrier()

            gather_start(0, 0)
            gather_start(1, 1)
            position(0, 0, 2, False)
            position(1, 1, 3, False)
            position(2, 2, 4, True)
            position(3, 3, 5, True)

            def gbody(g, _):
                c0 = g * 4
                for k in range(4):
                    position(c0 + k, k, c0 + k + 2, True)
                return 0

            lax.fori_loop(1, ngroups - 1, gbody, 0)
            c0 = (ngroups - 1) * 4
            position(c0 + 0, 0, c0 + 2, True)
            position(c0 + 1, 1, c0 + 3, True)
            position(c0 + 2, 2, None, False)
            position(c0 + 3, 3, None, False)
            for k in range(4):
                scatter_wait(c0 + k, k)
            plsc.subcore_barrier()
            pltpu.sync_copy(acc_sh.at[pl.ds(sid * nzr, nzr)],
                            out_hbm.at[pl.ds(tq * n + sid * nzr, nzr)])

    return agg


@functools.lru_cache(maxsize=None)
def _y1_kernel(n, fin, fo, po, nwp):
    nb = n // BN
    hn = fo // po

    def body(x_ref, w_ref, dn_ref, y_ref):
        dinv = 1.0 / jnp.sqrt(jnp.sum(dn_ref[...], axis=1, keepdims=True) + 1.0)
        y = jnp.dot(x_ref[...], w_ref[...],
                    preferred_element_type=jnp.float32) * dinv
        if po == 1:
            y_ref[...] = y
        else:
            for t in range(po):
                y_ref[t] = y[:, t * hn:(t + 1) * hn]

    return pl.pallas_call(
        body,
        grid=(nb,),
        in_specs=[
            pl.BlockSpec((BN, fin), lambda i: (i, 0)),
            pl.BlockSpec((fin, fo), lambda i: (0, 0)),
            pl.BlockSpec((BN, nwp), lambda i: (i, 0)),
        ],
        out_specs=(pl.BlockSpec((BN, fo), lambda i: (i, 0)) if po == 1 else
                   pl.BlockSpec((po, BN, hn), lambda i: (0, i, 0))),
        out_shape=jax.ShapeDtypeStruct(
            (n, fo) if po == 1 else (po, n, hn), jnp.float32),
    )


def _part_specs(nb, pi, hp):
    def mk(t):
        return pl.BlockSpec((BN, hp), lambda i, t=t: (i + t * nb, 0))

    return [mk(t) for t in range(pi)]


@functools.lru_cache(maxsize=None)
def _ymid_kernel(n, fp, fn, pi, po, nwp, esin=False):
    nb = n // BN
    hp = fp // pi
    hn = fn // po

    def body(*refs):
        a = refs[:pi + (1 if esin else 0)]
        o = len(a)
        y = refs[o:o + pi]
        dn, w = refs[o + pi], refs[o + pi + 1]
        gc = refs[o + pi + 2:o + 2 * pi + 2]
        bc = refs[o + 2 * pi + 2:o + 3 * pi + 2]
        yo = refs[o + 3 * pi + 2]
        dinv = 1.0 / jnp.sqrt(jnp.sum(dn[...], axis=1, keepdims=True) + 1.0)
        wv = w[...]
        acc = None
        for t in range(pi):
            at = a[t][...] + a[t + 1][...] if esin else a[t][...]
            ht = _leaky(gc[t][...] * (dinv * (at + y[t][...]))
                        + bc[t][...])
            d = jnp.dot(ht, wv[t * hp:(t + 1) * hp],
                        preferred_element_type=jnp.float32)
            acc = d if acc is None else acc + d
        yn = acc * dinv
        for t in range(po):
            yo[t] = yn[:, t * hn:(t + 1) * hn]

    return pl.pallas_call(
        body,
        grid=(nb,),
        in_specs=(
            _part_specs(nb, pi + (1 if esin else 0), hp)
            + _part_specs(nb, pi, hp)
            + [
                pl.BlockSpec((BN, nwp), lambda i: (i, 0)),
                pl.BlockSpec((fp, fn), lambda i: (0, 0)),
            ]
            + [pl.BlockSpec((1, hp), lambda i: (0, 0))] * (2 * pi)
        ),
        out_specs=pl.BlockSpec((po, BN, hn), lambda i: (0, i, 0)),
        out_shape=jax.ShapeDtypeStruct((po, n, hn), jnp.float32),
    )


@functools.lru_cache(maxsize=None)
def _head_kernel(n, f, pi, nwp):
    nb = n // BN
    hp = f // pi

    def body(*refs):
        a = refs[:pi]
        y = refs[pi:2 * pi]
        dn, b3 = refs[2 * pi], refs[2 * pi + 1]
        gc = refs[2 * pi + 2:3 * pi + 2]
        bc = refs[3 * pi + 2:4 * pi + 2]
        (fw1, fb1, fw2, fb2, fw3, fb3, fw4t, fb4,
         out) = refs[4 * pi + 2:4 * pi + 11]
        sums = refs[4 * pi + 11:5 * pi + 11]
        counts = refs[5 * pi + 11]
        i = pl.program_id(0)

        @pl.when(i == 0)
        def _():
            for t in range(pi):
                sums[t][...] = jnp.zeros_like(sums[t])
            counts[...] = jnp.zeros_like(counts)

        dinv = 1.0 / jnp.sqrt(jnp.sum(dn[...], axis=1, keepdims=True) + 1.0)
        bvec = b3[0]
        gids = lax.broadcasted_iota(jnp.int32, (NG, BN), 0)
        p = jnp.where(gids == jnp.broadcast_to(bvec, (NG, BN)), 1.0, 0.0)
        for t in range(pi):
            ht = _leaky(gc[t][...] * (dinv * (a[t][...] + y[t][...]))
                        + bc[t][...])
            sums[t][...] += jnp.dot(p, ht,
                                    preferred_element_type=jnp.float32,
                                    precision=lax.Precision.HIGHEST)
        counts[...] += jnp.dot(p, jnp.ones((BN, hp), jnp.float32),
                               preferred_element_type=jnp.float32,
                               precision=lax.Precision.HIGHEST)

        @pl.when(i == nb - 1)
        def _():
            cnt = jnp.maximum(counts[...], 1.0)
            w1 = fw1[...]
            acc = None
            for t in range(pi):
                d = jnp.dot(sums[t][...] / cnt, w1[t * hp:(t + 1) * hp],
                            preferred_element_type=jnp.float32)
                acc = d if acc is None else acc + d
            a1 = _leaky(acc + fb1[...])
            a2 = _leaky(jnp.dot(a1, fw2[...],
                                preferred_element_type=jnp.float32) + fb2[...])
            a3 = _leaky(jnp.dot(a2, fw3[...],
                                preferred_element_type=jnp.float32) + fb3[...])
            r = jnp.dot(a3, fw4t[...],
                        preferred_element_type=jnp.float32) + fb4[...]
            out[...] = jnp.broadcast_to(r, (NG, 128))

    return pl.pallas_call(
        body,
        grid=(nb,),
        in_specs=(
            _part_specs(nb, pi, hp) * 2
            + [
                pl.BlockSpec((BN, nwp), lambda i: (i, 0)),
                pl.BlockSpec((1, 1, BN), lambda i: (i, 0, 0)),
            ]
            + [pl.BlockSpec((1, hp), lambda i: (0, 0))] * (2 * pi)
            + [
                pl.BlockSpec((128, 64), lambda i: (0, 0)),
                pl.BlockSpec((1, 64), lambda i: (0, 0)),
                pl.BlockSpec((64, 32), lambda i: (0, 0)),
                pl.BlockSpec((1, 32), lambda i: (0, 0)),
                pl.BlockSpec((32, 16), lambda i: (0, 0)),
                pl.BlockSpec((1, 16), lambda i: (0, 0)),
                pl.BlockSpec((16, 1), lambda i: (0, 0)),
                pl.BlockSpec((1, 1), lambda i: (0, 0)),
            ]
        ),
        out_specs=pl.BlockSpec((NG, 128), lambda i: (0, 0)),
        out_shape=jax.ShapeDtypeStruct((NG, 128), jnp.float32),
        scratch_shapes=[pltpu.VMEM((NG, hp), jnp.float32)] * (pi + 1),
    )


_P1, _P2, _P3 = 2, 2, 4


def kernel(x, edge_index, edge_weight, batch, W1, b1, W2, b2, W3, b3,
           g1, be1, g2, be2, g3, be3,
           fW1, fb1, fW2, fb2, fW3, fb3, fW4, fb4):
    n, fin = x.shape
    e = edge_weight.shape[0]
    src = edge_index[0].astype(jnp.int32)
    dst = edge_index[1].astype(jnp.int32)
    ew = edge_weight.astype(jnp.float32)

    nchunk = -4 * (-e // (4 * NS * C))
    ep = NS * C * nchunk
    pad = ep - e
    if pad:
        src = jnp.concatenate([src, jnp.zeros((pad,), jnp.int32)])
        dst = jnp.concatenate([dst, jnp.zeros((pad,), jnp.int32)])
        ew = jnp.concatenate([ew, jnp.zeros((pad,), jnp.float32)])
    src2 = src.reshape(-1, C)
    dst2 = dst.reshape(-1, C)

    epd = -NW * 16 * (-e // (NW * 16))
    padd = epd - e
    dstd = jnp.concatenate([dst[:e], jnp.zeros((padd,), jnp.int32)])
    ewd = jnp.concatenate([ew[:e], jnp.zeros((padd,), jnp.float32)])
    deg_parts = _deg_kernel(n, epd)(dstd, ewd)
    dn = deg_parts.T

    s = 1.0 / jnp.sqrt(1.0 + EPS)

    def split(v, parts):
        hh = v.shape[0] // parts
        return tuple(v[t * hh:(t + 1) * hh][None, :] for t in range(parts))

    g1c = split(g1 * s, _P1)
    b1c = split(g1 * s * b1 + be1, _P1)
    g2c = split(g2 * s, _P2)
    del g1c, b1c
    g1cf = (g1 * s)[None, :]
    b1cf = (g1 * s * b1 + be1)[None, :]
    b2c = split(g2 * s * b2 + be2, _P2)
    g3c = split(g3 * s, _P3)
    b3c = split(g3 * s * b3 + be3, _P3)

    f1, f2, f3 = W1.shape[1], W2.shape[1], W3.shape[1]
    z1 = jnp.zeros((n, f1), jnp.float32)
    z2 = jnp.zeros((n, f2 // _P2), jnp.float32)
    z3 = jnp.zeros((n, f3 // _P3), jnp.float32)

    y1 = _y1_kernel(n, fin, f1, 1, NW)(x, W1, dn)
    a1 = _agg_kernel(n, ep, f1, 1)(y1, src2, dst2, ew, z1)
    y2 = _ymid_kernel(n, f1, f2, 1, _P2, NW, True)(
        a1, a1, y1, dn, W2, g1cf, b1cf
    ).reshape(_P2 * n, f2 // _P2)
    a2 = _agg_kernel(n, ep, f2 // _P2, _P2)(y2, src2, dst2, ew, z2)
    y3 = _ymid_kernel(n, f2, f3, _P2, _P3, NW)(
        *([a2] * _P2), *([y2] * _P2), dn, W3, *g2c, *b2c
    ).reshape(_P3 * n, f3 // _P3)
    a3 = _agg_kernel(n, ep, f3 // _P3, _P3)(y3, src2, dst2, ew, z3)

    nb = n // BN
    batch3 = batch.astype(jnp.int32).reshape(nb, 1, BN)
    head = _head_kernel(n, f3, _P3, NW)(
        *([a3] * _P3), *([y3] * _P3), dn, batch3, *g3c, *b3c,
        fW1, fb1[None, :], fW2, fb2[None, :], fW3, fb3[None, :],
        fW4, fb4[None, :])
    return head[:, 0]


# speedup vs baseline: 1.3880x; 1.0006x over previous
import functools

import jax
import jax.numpy as jnp
from jax import lax
from jax.experimental import pallas as pl
from jax.experimental.pallas import tpu as pltpu
from jax.experimental.pallas import tpu_sc as plsc

NC = 2
NS = 16
NW = NC * NS
C = 128
SLOPE = 0.01
EPS = 1e-5
NG = 64
BN = 2000


def _leaky(h):
    return jnp.where(h >= 0, h, SLOPE * h)


def _mesh():
    return plsc.VectorSubcoreMesh(core_axis_name="c", subcore_axis_name="s")


_SC_PARAMS = dict(
    compiler_params=pltpu.CompilerParams(
        needs_layout_passes=False, use_tc_tiling_on_sc=False),
)


@functools.lru_cache(maxsize=None)
def _deg_kernel(n, ep):
    epw = ep // NW

    @functools.partial(
        pl.kernel,
        out_type=jax.ShapeDtypeStruct((NW, n), jnp.float32),
        mesh=_mesh(),
        scratch_types=[
            pltpu.VMEM((epw,), jnp.int32),
            pltpu.VMEM((epw,), jnp.float32),
            pltpu.VMEM((n,), jnp.float32),
        ],
        **_SC_PARAMS,
    )
    def deg(dst_hbm, ew_hbm, out_hbm, dst_v, ew_v, acc_v):
        cid = lax.axis_index("c")
        sid = lax.axis_index("s")
        wid = cid * NS + sid
        z = jnp.zeros((16,), jnp.float32)

        def zbody(i, _):
            acc_v[pl.ds(i * 16, 16)] = z
            return 0

        lax.fori_loop(0, n // 16, zbody, 0, unroll=8)
        pltpu.sync_copy(dst_hbm.at[pl.ds(wid * epw, epw)], dst_v)
        pltpu.sync_copy(ew_hbm.at[pl.ds(wid * epw, epw)], ew_v)

        def body(i, _):
            idx = dst_v[pl.ds(i * 16, 16)]
            w = ew_v[pl.ds(i * 16, 16)]
            plsc.addupdate_scatter(acc_v, [idx], w)
            return 0

        lax.fori_loop(0, epw // 16, body, 0, unroll=4)
        pltpu.sync_copy(acc_v, out_hbm.at[wid])

    return deg


@functools.lru_cache(maxsize=None)
def _agg_kernel(n, ep, h, parts):
    edge_split = parts == 1
    npass = max(parts // 2, 1)
    ntile = NW if edge_split else NS
    nchunk = ep // (ntile * C)
    assert nchunk % 4 == 0 and nchunk >= 8
    ngroups = nchunk // 4
    ept = ep // ntile
    nzr = n // NS
    unroll = 4 if h >= 64 else 8
    nout = 2 if edge_split else parts

    @functools.partial(
        pl.kernel,
        out_type=jax.ShapeDtypeStruct((nout * n, h), jnp.float32),
        mesh=_mesh(),
        scratch_types=[
            pltpu.VMEM((nchunk, C), jnp.int32),
            pltpu.VMEM((nchunk, C), jnp.int32),
            pltpu.VMEM((ept,), jnp.float32),
            pltpu.VMEM((C, h), jnp.float32),
            pltpu.VMEM((C, h), jnp.float32),
            pltpu.VMEM((C, h), jnp.float32),
            pltpu.VMEM((C, h), jnp.float32),
            pltpu.VMEM_SHARED((n, h), jnp.float32),
            pltpu.VMEM_SHARED((n, h), jnp.float32),
            pltpu.SemaphoreType.DMA,
            pltpu.SemaphoreType.DMA,
            pltpu.SemaphoreType.DMA,
            pltpu.SemaphoreType.DMA,
            pltpu.SemaphoreType.DMA,
            pltpu.SemaphoreType.DMA,
            pltpu.SemaphoreType.DMA,
            pltpu.SemaphoreType.DMA,
        ],
        **_SC_PARAMS,
    )
    def agg(y_hbm, src_hbm, dst_hbm, ew_hbm, zeros_hbm, out_hbm,
            src_v, dst_v, ew_v, r0, r1, r2, r3, acc_sh, ystage,
            g0, g1, g2, g3, s0, s1, s2, s3):
        rows = [r0, r1, r2, r3]
        gsem = [g0, g1, g2, g3]
        ssem = [s0, s1, s2, s3]
        cid = lax.axis_index("c")
        sid = lax.axis_index("s")
        tix = cid * NS + sid if edge_split else sid
        pltpu.sync_copy(src_hbm.at[pl.ds(tix * nchunk, nchunk)], src_v)
        pltpu.sync_copy(dst_hbm.at[pl.ds(tix * nchunk, nchunk)], dst_v)
        pltpu.sync_copy(ew_hbm.at[pl.ds(tix * ept, ept)], ew_v)

        def gather_start(c, b):
            pltpu.make_async_copy(ystage.at[src_v.at[c]], rows[b],
                                  gsem[b]).start()

        def gather_wait(c, b):
            pltpu.make_async_copy(ystage.at[src_v.at[c]], rows[b],
                                  gsem[b]).wait()

        def scatter_start(c, b):
            pltpu.make_async_copy(rows[b], acc_sh.at[dst_v.at[c]],
                                  ssem[b]).start(add=True)

        def scatter_wait(c, b):
            pltpu.make_async_copy(rows[b], acc_sh.at[dst_v.at[c]],
                                  ssem[b]).wait()

        def multiply(c, b):
            rbuf = rows[b]

            def mbody(e, _):
                w = plsc.load_gather(
                    ew_v, [jnp.full((16,), c * C + e, jnp.int32)])
                for j in range(h // 16):
                    sl = pl.ds(j * 16, 16)
                    rbuf[e, sl] = rbuf[e, sl] * w
                return 0

            lax.fori_loop(0, C, mbody, 0, unroll=unroll)

        def position(c, k, pre_c, drain):
            pb = (k + 2) % 4
            if pre_c is not None:
                if drain:
                    scatter_wait(pre_c - 4, pb)
                gather_start(pre_c, pb)
            gather_wait(c, k)
            multiply(c, k)
            scatter_start(c, k)

        for q in range(npass):
            tq = cid * npass + q
            ty = 0 if edge_split else tq
            pltpu.sync_copy(zeros_hbm.at[pl.ds(sid * nzr, nzr)],
                            acc_sh.at[pl.ds(sid * nzr, nzr)])
            pltpu.sync_copy(y_hbm.at[pl.ds(ty * n + sid * nzr, nzr)],
                            ystage.at[pl.ds(sid * nzr, nzr)])
            plsc.subcore_barrier()

            gather_start(0, 0)
            gather_start(1, 1)
            position(0, 0, 2, False)
            position(1, 1, 3, False)
            position(2, 2, 4, True)
            position(3, 3, 5, True)

            def gbody(g, _):
                c0 = g * 4
                for k in range(4):
                    position(c0 + k, k, c0 + k + 2, True)
                return 0

            lax.fori_loop(1, ngroups - 1, gbody, 0)
            c0 = (ngroups - 1) * 4
            position(c0 + 0, 0, c0 + 2, True)
            position(c0 + 1, 1, c0 + 3, True)
            position(c0 + 2, 2, None, False)
            position(c0 + 3, 3, None, False)
            for k in range(4):
                scatter_wait(c0 + k, k)
            plsc.subcore_barrier()
            pltpu.sync_copy(acc_sh.at[pl.ds(sid * nzr, nzr)],
                            out_hbm.at[pl.ds(tq * n + sid * nzr, nzr)])

    return agg


@functools.lru_cache(maxsize=None)
def _y1_kernel(n, fin, fo, po, nwp):
    nb = n // BN
    hn = fo // po

    def body(x_ref, w_ref, dn_ref, y_ref):
        dinv = 1.0 / jnp.sqrt(jnp.sum(dn_ref[...], axis=1, keepdims=True) + 1.0)
        y = jnp.dot(x_ref[...], w_ref[...],
                    preferred_element_type=jnp.float32) * dinv
        if po == 1:
            y_ref[...] = y
        else:
            for t in range(po):
                y_ref[t] = y[:, t * hn:(t + 1) * hn]

    return pl.pallas_call(
        body,
        grid=(nb,),
        in_specs=[
            pl.BlockSpec((BN, fin), lambda i: (i, 0)),
            pl.BlockSpec((fin, fo), lambda i: (0, 0)),
            pl.BlockSpec((BN, nwp), lambda i: (i, 0)),
        ],
        out_specs=(pl.BlockSpec((BN, fo), lambda i: (i, 0)) if po == 1 else
                   pl.BlockSpec((po, BN, hn), lambda i: (0, i, 0))),
        out_shape=jax.ShapeDtypeStruct(
            (n, fo) if po == 1 else (po, n, hn), jnp.float32),
    )


def _part_specs(nb, pi, hp):
    def mk(t):
        return pl.BlockSpec((BN, hp), lambda i, t=t: (i + t * nb, 0))

    return [mk(t) for t in range(pi)]


@functools.lru_cache(maxsize=None)
def _ymid_kernel(n, fp, fn, pi, po, nwp, esin=False):
    nb = n // BN
    hp = fp // pi
    hn = fn // po

    def body(*refs):
        a = refs[:pi + (1 if esin else 0)]
        o = len(a)
        y = refs[o:o + pi]
        dn, w = refs[o + pi], refs[o + pi + 1]
        gc = refs[o + pi + 2:o + 2 * pi + 2]
        bc = refs[o + 2 * pi + 2:o + 3 * pi + 2]
        yo = refs[o + 3 * pi + 2]
        dinv = 1.0 / jnp.sqrt(jnp.sum(dn[...], axis=1, keepdims=True) + 1.0)
        wv = w[...]
        acc = None
        for t in range(pi):
            at = a[t][...] + a[t + 1][...] if esin else a[t][...]
            ht = _leaky(gc[t][...] * (dinv * (at + y[t][...]))
                        + bc[t][...])
            d = jnp.dot(ht, wv[t * hp:(t + 1) * hp],
                        preferred_element_type=jnp.float32)
            acc = d if acc is None else acc + d
        yn = acc * dinv
        for t in range(po):
            yo[t] = yn[:, t * hn:(t + 1) * hn]

    return pl.pallas_call(
        body,
        grid=(nb,),
        in_specs=(
            _part_specs(nb, pi + (1 if esin else 0), hp)
            + _part_specs(nb, pi, hp)
            + [
                pl.BlockSpec((BN, nwp), lambda i: (i, 0)),
                pl.BlockSpec((fp, fn), lambda i: (0, 0)),
            ]
            + [pl.BlockSpec((1, hp), lambda i: (0, 0))] * (2 * pi)
        ),
        out_specs=pl.BlockSpec((po, BN, hn), lambda i: (0, i, 0)),
        out_shape=jax.ShapeDtypeStruct((po, n, hn), jnp.float32),
    )


@functools.lru_cache(maxsize=None)
def _head_kernel(n, f, pi, nwp):
    nb = n // BN
    hp = f // pi

    def body(*refs):
        a = refs[:pi]
        y = refs[pi:2 * pi]
        dn, b3 = refs[2 * pi], refs[2 * pi + 1]
        gc = refs[2 * pi + 2:3 * pi + 2]
        bc = refs[3 * pi + 2:4 * pi + 2]
        (fw1, fb1, fw2, fb2, fw3, fb3, fw4t, fb4,
         out) = refs[4 * pi + 2:4 * pi + 11]
        sums = refs[4 * pi + 11:5 * pi + 11]
        counts = refs[5 * pi + 11]
        i = pl.program_id(0)

        @pl.when(i == 0)
        def _():
            for t in range(pi):
                sums[t][...] = jnp.zeros_like(sums[t])
            counts[...] = jnp.zeros_like(counts)

        dinv = 1.0 / jnp.sqrt(jnp.sum(dn[...], axis=1, keepdims=True) + 1.0)
        bvec = b3[0]
        gids = lax.broadcasted_iota(jnp.int32, (NG, BN), 0)
        p = jnp.where(gids == jnp.broadcast_to(bvec, (NG, BN)), 1.0, 0.0)
        for t in range(pi):
            ht = _leaky(gc[t][...] * (dinv * (a[t][...] + y[t][...]))
                        + bc[t][...])
            sums[t][...] += jnp.dot(p, ht,
                                    preferred_element_type=jnp.float32,
                                    precision=lax.Precision.HIGHEST)
        counts[...] += jnp.dot(p, jnp.ones((BN, hp), jnp.float32),
                               preferred_element_type=jnp.float32,
                               precision=lax.Precision.HIGHEST)

        @pl.when(i == nb - 1)
        def _():
            cnt = jnp.maximum(counts[...], 1.0)
            w1 = fw1[...]
            acc = None
            for t in range(pi):
                d = jnp.dot(sums[t][...] / cnt, w1[t * hp:(t + 1) * hp],
                            preferred_element_type=jnp.float32)
                acc = d if acc is None else acc + d
            a1 = _leaky(acc + fb1[...])
            a2 = _leaky(jnp.dot(a1, fw2[...],
                                preferred_element_type=jnp.float32) + fb2[...])
            a3 = _leaky(jnp.dot(a2, fw3[...],
                                preferred_element_type=jnp.float32) + fb3[...])
            r = jnp.dot(a3, fw4t[...],
                        preferred_element_type=jnp.float32) + fb4[...]
            out[...] = jnp.broadcast_to(r, (NG, 128))

    return pl.pallas_call(
        body,
        grid=(nb,),
        in_specs=(
            _part_specs(nb, pi, hp) * 2
            + [
                pl.BlockSpec((BN, nwp), lambda i: (i, 0)),
                pl.BlockSpec((1, 1, BN), lambda i: (i, 0, 0)),
            ]
            + [pl.BlockSpec((1, hp), lambda i: (0, 0))] * (2 * pi)
            + [
                pl.BlockSpec((128, 64), lambda i: (0, 0)),
                pl.BlockSpec((1, 64), lambda i: (0, 0)),
                pl.BlockSpec((64, 32), lambda i: (0, 0)),
                pl.BlockSpec((1, 32), lambda i: (0, 0)),
                pl.BlockSpec((32, 16), lambda i: (0, 0)),
                pl.BlockSpec((1, 16), lambda i: (0, 0)),
                pl.BlockSpec((16, 1), lambda i: (0, 0)),
                pl.BlockSpec((1, 1), lambda i: (0, 0)),
            ]
        ),
        out_specs=pl.BlockSpec((NG, 128), lambda i: (0, 0)),
        out_shape=jax.ShapeDtypeStruct((NG, 128), jnp.float32),
        scratch_shapes=[pltpu.VMEM((NG, hp), jnp.float32)] * (pi + 1),
    )


_P2, _P3 = 2, 4


def kernel(x, edge_index, edge_weight, batch, W1, b1, W2, b2, W3, b3,
           g1, be1, g2, be2, g3, be3,
           fW1, fb1, fW2, fb2, fW3, fb3, fW4, fb4):
    n, fin = x.shape
    e = edge_weight.shape[0]
    src = edge_index[0].astype(jnp.int32)
    dst = edge_index[1].astype(jnp.int32)
    ew = edge_weight.astype(jnp.float32)

    nchunk = -4 * (-e // (4 * NS * C))
    ep = NS * C * nchunk
    pad = ep - e
    if pad:
        src = jnp.concatenate([src, jnp.zeros((pad,), jnp.int32)])
        dst = jnp.concatenate([dst, jnp.zeros((pad,), jnp.int32)])
        ew = jnp.concatenate([ew, jnp.zeros((pad,), jnp.float32)])
    src2 = src.reshape(-1, C)
    dst2 = dst.reshape(-1, C)

    epd = -NW * 16 * (-e // (NW * 16))
    padd = epd - e
    dstd = jnp.concatenate([dst[:e], jnp.zeros((padd,), jnp.int32)])
    ewd = jnp.concatenate([ew[:e], jnp.zeros((padd,), jnp.float32)])
    deg_parts = _deg_kernel(n, epd)(dstd, ewd)
    dn = deg_parts.T

    s = 1.0 / jnp.sqrt(1.0 + EPS)

    def split(v, parts):
        hh = v.shape[0] // parts
        return tuple(v[t * hh:(t + 1) * hh][None, :] for t in range(parts))

    g1cf = (g1 * s)[None, :]
    b1cf = (g1 * s * b1 + be1)[None, :]
    g2c = split(g2 * s, _P2)
    b2c = split(g2 * s * b2 + be2, _P2)
    g3c = split(g3 * s, _P3)
    b3c = split(g3 * s * b3 + be3, _P3)

    f1, f2, f3 = W1.shape[1], W2.shape[1], W3.shape[1]
    z1 = jnp.zeros((n, f1), jnp.float32)
    z2 = jnp.zeros((n, f2 // _P2), jnp.float32)
    z3 = jnp.zeros((n, f3 // _P3), jnp.float32)

    y1 = _y1_kernel(n, fin, f1, 1, NW)(x, W1, dn)
    a1 = _agg_kernel(n, ep, f1, 1)(y1, src2, dst2, ew, z1)
    y2 = _ymid_kernel(n, f1, f2, 1, _P2, NW, True)(
        a1, a1, y1, dn, W2, g1cf, b1cf
    ).reshape(_P2 * n, f2 // _P2)
    a2 = _agg_kernel(n, ep, f2 // _P2, _P2)(y2, src2, dst2, ew, z2)
    y3 = _ymid_kernel(n, f2, f3, _P2, _P3, NW)(
        *([a2] * _P2), *([y2] * _P2), dn, W3, *g2c, *b2c
    ).reshape(_P3 * n, f3 // _P3)
    a3 = _agg_kernel(n, ep, f3 // _P3, _P3)(y3, src2, dst2, ew, z3)

    nb = n // BN
    batch3 = batch.astype(jnp.int32).reshape(nb, 1, BN)
    head = _head_kernel(n, f3, _P3, NW)(
        *([a3] * _P3), *([y3] * _P3), dn, batch3, *g3c, *b3c,
        fW1, fb1[None, :], fW2, fb2[None, :], fW3, fb3[None, :],
        fW4, fb4[None, :])
    return head[:, 0]
